# baseline (speedup 1.0000x reference)
import functools

import jax
import jax.numpy as jnp
from jax import lax
from jax.experimental import pallas as pl
from jax.experimental.pallas import tpu as pltpu
from jax.experimental.pallas import tpu_sc as plsc

B = 4096
D = 32
H = 50
NUM_BUCKETS = 500000
NC = 2
NS = 16
NW = NC * NS
BPW = B // NW
G = 4
NGROUPS = BPW // G
L = 16


def _sc_gather_features(did, vid, watch, ctx, pos, uemb, vemb, ctx_q, ctx_r,
                        pos_emb):
  mesh = plsc.VectorSubcoreMesh(
      core_axis_name="c", subcore_axis_name="s", num_cores=NC,
      num_subcores=NS)

  @functools.partial(
      pl.kernel,
      mesh=mesh,
      out_type=jax.ShapeDtypeStruct((5, B, D), jnp.float32),
      compiler_params=pltpu.CompilerParams(use_tc_tiling_on_sc=False),
      scratch_types=[
          pltpu.VMEM((BPW,), jnp.int32),
          pltpu.VMEM((BPW,), jnp.int32),
          pltpu.VMEM((BPW,), jnp.int32),
          pltpu.VMEM((BPW,), jnp.int32),
          pltpu.VMEM((BPW,), jnp.int32),
          pltpu.VMEM((BPW,), jnp.int32),
          pltpu.VMEM((BPW, H), jnp.int32),
          pltpu.VMEM((BPW, D), jnp.float32),
          pltpu.VMEM((BPW, D), jnp.float32),
          pltpu.VMEM((BPW, D), jnp.float32),
          pltpu.VMEM((BPW, D), jnp.float32),
          pltpu.VMEM((BPW, D), jnp.float32),
          pltpu.VMEM((BPW, D), jnp.float32),
          pltpu.VMEM((2, G, H, D), jnp.float32),
          pltpu.SemaphoreType.DMA,
          pltpu.SemaphoreType.DMA,
          pltpu.SemaphoreType.DMA,
          pltpu.SemaphoreType.DMA,
          pltpu.SemaphoreType.DMA,
          pltpu.SemaphoreType.DMA,
          pltpu.SemaphoreType.DMA,
      ],
  )
  def sc_fn(did_h, vid_h, watch_h, ctx_h, pos_h, uemb_h, vemb_h, ctxq_h,
            ctxr_h, pemb_h, out_h, did_v, vid_v, ctx_v, pos_v, q_v, r_v,
            widx_v, u_v, v_v, cq_v, cr_v, p_v, his_v, rows_v, sem_u, sem_v,
            sem_cq, sem_cr, sem_p, sem_h0, sem_h1):
    wid = lax.axis_index("s") * NC + lax.axis_index("c")
    base = wid * BPW
    hsems = (sem_h0, sem_h1)

    pltpu.sync_copy(did_h.at[pl.ds(base, BPW)], did_v)
    pltpu.sync_copy(vid_h.at[pl.ds(base, BPW)], vid_v)
    pltpu.sync_copy(ctx_h.at[pl.ds(base, BPW)], ctx_v)
    pltpu.sync_copy(pos_h.at[pl.ds(base, BPW)], pos_v)
    pltpu.sync_copy(watch_h.at[pl.ds(base, BPW), :], widx_v)

    nb = jnp.full((L,), NUM_BUCKETS, jnp.int32)
    for j in range(BPW // L):
      c = ctx_v[pl.ds(j * L, L)]
      q_v[pl.ds(j * L, L)] = lax.div(c, nb)
      r_v[pl.ds(j * L, L)] = lax.rem(c, nb)

    pltpu.async_copy(uemb_h.at[did_v], u_v, sem_u)
    pltpu.async_copy(vemb_h.at[vid_v], v_v, sem_v)
    pltpu.async_copy(ctxq_h.at[q_v], cq_v, sem_cq)
    pltpu.async_copy(ctxr_h.at[r_v], cr_v, sem_cr)
    pltpu.async_copy(pemb_h.at[pos_v], p_v, sem_p)

    def fire(g, buf):
      for k in range(G):
        pltpu.async_copy(vemb_h.at[widx_v.at[g * G + k]], rows_v.at[buf, k],
                         hsems[buf])

    def drain(g, buf):
      for k in range(G):
        pltpu.make_async_copy(vemb_h.at[widx_v.at[g * G + k]],
                              rows_v.at[buf, k], hsems[buf]).wait()

    def reduce(g, buf):
      for k in range(G):
        a0 = rows_v[buf, k, 0, pl.ds(0, L)]
        a1 = rows_v[buf, k, 0, pl.ds(L, L)]
        for h in range(1, H):
          a0 = a0 + rows_v[buf, k, h, pl.ds(0, L)]
          a1 = a1 + rows_v[buf, k, h, pl.ds(L, L)]
        item = g * G + k
        his_v[item, pl.ds(0, L)] = a0 * (1.0 / H)
        his_v[item, pl.ds(L, L)] = a1 * (1.0 / H)

    fire(0, 0)

    def outer(step, carry):
      for b in range(2):
        g = step * 2 + b

        @pl.when(g + 1 < NGROUPS)
        def _():
          fire(g + 1, 1 - b)

        drain(g, b)
        reduce(g, b)
      return carry

    lax.fori_loop(0, NGROUPS // 2, outer, 0)

    pltpu.make_async_copy(uemb_h.at[did_v], u_v, sem_u).wait()
    pltpu.make_async_copy(vemb_h.at[vid_v], v_v, sem_v).wait()
    pltpu.make_async_copy(ctxq_h.at[q_v], cq_v, sem_cq).wait()
    pltpu.make_async_copy(ctxr_h.at[r_v], cr_v, sem_cr).wait()
    pltpu.make_async_copy(pemb_h.at[pos_v], p_v, sem_p).wait()

    for i in range(BPW):
      for c in range(D // L):
        cq_v[i, pl.ds(c * L, L)] = (cq_v[i, pl.ds(c * L, L)] *
                                    cr_v[i, pl.ds(c * L, L)])

    pltpu.sync_copy(u_v, out_h.at[0, pl.ds(base, BPW), :])
    pltpu.sync_copy(v_v, out_h.at[1, pl.ds(base, BPW), :])
    pltpu.sync_copy(his_v, out_h.at[2, pl.ds(base, BPW), :])
    pltpu.sync_copy(cq_v, out_h.at[3, pl.ds(base, BPW), :])
    pltpu.sync_copy(p_v, out_h.at[4, pl.ds(base, BPW), :])

  return sc_fn(did, vid, watch, ctx, pos, uemb, vemb, ctx_q, ctx_r, pos_emb)


def _mlp_body(x_ref, w1_ref, b1_ref, w2_ref, b2_ref, w3_ref, b3_ref, wo_ref,
              bo_ref, o_ref):
  acc = jnp.dot(x_ref[0], w1_ref[0], preferred_element_type=jnp.float32)
  for f in range(1, 5):
    acc = acc + jnp.dot(x_ref[f], w1_ref[f],
                        preferred_element_type=jnp.float32)
  h1 = jnp.maximum(acc + b1_ref[...], 0.0)
  h2 = jnp.maximum(
      jnp.dot(h1, w2_ref[...], preferred_element_type=jnp.float32)
      + b2_ref[...], 0.0)
  h3 = jnp.maximum(
      jnp.dot(h2, w3_ref[...], preferred_element_type=jnp.float32)
      + b3_ref[...], 0.0)
  o_ref[...] = (jnp.dot(h3, wo_ref[...], preferred_element_type=jnp.float32)
                + bo_ref[...])


def _mlp(x, W1, b1, W2, b2, W3, b3, Wo, bo):
  bm = 512
  grid = (B // bm,)
  return pl.pallas_call(
      _mlp_body,
      grid=grid,
      in_specs=[
          pl.BlockSpec((5, bm, D), lambda i: (0, i, 0)),
          pl.BlockSpec((5, D, 512), lambda i: (0, 0, 0)),
          pl.BlockSpec((1, 512), lambda i: (0, 0)),
          pl.BlockSpec((512, 256), lambda i: (0, 0)),
          pl.BlockSpec((1, 256), lambda i: (0, 0)),
          pl.BlockSpec((256, 64), lambda i: (0, 0)),
          pl.BlockSpec((1, 64), lambda i: (0, 0)),
          pl.BlockSpec((64, 1), lambda i: (0, 0)),
          pl.BlockSpec((1, 1), lambda i: (0, 0)),
      ],
      out_specs=pl.BlockSpec((bm, 1), lambda i: (i, 0)),
      out_shape=jax.ShapeDtypeStruct((B, 1), jnp.float32),
  )(x, W1, b1, W2, b2, W3, b3, Wo, bo)


def kernel(did, vid, watch_vids, context_id, position, uemb, vemb, ctx_q,
           ctx_r, pos_emb, W1, b1, W2, b2, W3, b3, Wo, bo):
  feats = _sc_gather_features(
      did.astype(jnp.int32), vid.astype(jnp.int32),
      watch_vids.astype(jnp.int32), context_id.astype(jnp.int32),
      position.astype(jnp.int32), uemb, vemb, ctx_q, ctx_r, pos_emb)
  return _mlp(feats, W1.reshape(5, D, 512), b1.reshape(1, 512), W2,
              b2.reshape(1, 256), W3, b3.reshape(1, 64), Wo,
              bo.reshape(1, 1))
